# Initial kernel scaffold; baseline (speedup 1.0000x reference)
import jax
import jax.numpy as jnp
from jax.experimental import pallas as pl


def kernel(x, edge_index, batch, W1, a_s1, a_d1, b1, W2, a_s2, a_d2, b2, W3, a_s3, a_d3, b3, fc1_W, fc1_b, g4, be4, fc2_W, fc2_b, g5, be5):
    raise NotImplementedError("write your pallas kernel here")


# speedup vs baseline: 1.2272x; 1.2272x over previous
import functools

import jax
import jax.numpy as jnp
from jax import lax
from jax.experimental import pallas as pl
from jax.experimental.pallas import tpu as pltpu
from jax.experimental.pallas import tpu_sc as plsc

N = 10000
E = 320000
G = 64
F = 128
NC = 2
NS = 16
EPW = E // (NC * NS)
K = 80
NB = EPW // K
NPAD = 10240
RPT = NPAD // NS
RCH = 128


def _lin_attn_body(x_ref, W_ref, as_ref, ad_ref, h_ref, hs_ref, hd_ref):
    h = jnp.dot(x_ref[...], W_ref[...])
    h_ref[...] = h
    hs_ref[...] = jnp.dot(h, as_ref[...])
    hd_ref[...] = jnp.dot(h, ad_ref[...])


def _lin_attn(x, W, a_s, a_d):
    dout = W.shape[1]
    h, hs, hd = pl.pallas_call(
        _lin_attn_body,
        out_shape=(
            jax.ShapeDtypeStruct((N, dout), jnp.float32),
            jax.ShapeDtypeStruct((N, 1), jnp.float32),
            jax.ShapeDtypeStruct((N, 1), jnp.float32),
        ),
    )(x, W, a_s.reshape(-1, 1), a_d.reshape(-1, 1))
    return h, hs[:, 0], hd[:, 0]


def _epi_lin_attn_body(p0_ref, p1_ref, b_ref, W_ref, as_ref, ad_ref,
                       h_ref, hs_ref, hd_ref):
    hin = jax.nn.relu(p0_ref[...] + p1_ref[...] + b_ref[...])
    h = jnp.dot(hin, W_ref[...])
    h_ref[...] = h
    hs_ref[...] = jnp.dot(h, as_ref[...])
    hd_ref[...] = jnp.dot(h, ad_ref[...])


def _epi_lin_attn(p0, p1, b, W, a_s, a_d):
    dout = W.shape[1]
    h, hs, hd = pl.pallas_call(
        _epi_lin_attn_body,
        out_shape=(
            jax.ShapeDtypeStruct((N, dout), jnp.float32),
            jax.ShapeDtypeStruct((N, 1), jnp.float32),
            jax.ShapeDtypeStruct((N, 1), jnp.float32),
        ),
    )(p0, p1, b.reshape(1, -1), W, a_s.reshape(-1, 1), a_d.reshape(-1, 1))
    return h, hs[:, 0], hd[:, 0]


def _mlp_body(p_ref, fc1_W_ref, fc1_b_ref, g4_ref, be4_ref, fc2_W_ref,
              fc2_b_ref, g5_ref, be5_ref, out_ref):
    z = jnp.dot(p_ref[...], fc1_W_ref[...]) + fc1_b_ref[...]
    mu = jnp.mean(z, axis=0, keepdims=True)
    var = jnp.mean((z - mu) ** 2, axis=0, keepdims=True)
    z = (z - mu) * lax.rsqrt(var + 1e-5) * g4_ref[...] + be4_ref[...]
    z = jax.nn.relu(z)
    o = jnp.dot(z, fc2_W_ref[...]) + fc2_b_ref[...]
    mu2 = jnp.mean(o, axis=0, keepdims=True)
    var2 = jnp.mean((o - mu2) ** 2, axis=0, keepdims=True)
    out_ref[...] = (o - mu2) * lax.rsqrt(var2 + 1e-5) * g5_ref[...] + be5_ref[...]


def _mlp(p, fc1_W, fc1_b, g4, be4, fc2_W, fc2_b, g5, be5):
    return pl.pallas_call(
        _mlp_body,
        out_shape=jax.ShapeDtypeStruct((G, 256), jnp.float32),
    )(p, fc1_W, fc1_b.reshape(1, -1), g4.reshape(1, -1), be4.reshape(1, -1),
      fc2_W, fc2_b.reshape(1, -1), g5.reshape(1, -1), be5.reshape(1, -1))


def _rowsum_body(src_hbm, dst_hbm, alpha_hbm, h_hbm, out_hbm,
                 accum_sh, sidx, didx, arows, rows, tmp, sem):
    cid = lax.axis_index("c")
    sid = lax.axis_index("s")
    wid = sid * NC + cid

    def zrow(r, _):
        for f in range(F // 16):
            tmp[r, pl.ds(f * 16, 16)] = jnp.zeros((16,), jnp.float32)
        return 0
    lax.fori_loop(0, RCH, zrow, 0)
    for cpy in range(RPT // RCH):
        pltpu.sync_copy(tmp, accum_sh.at[pl.ds(sid * RPT + cpy * RCH, RCH)])
    plsc.subcore_barrier()

    base = wid * EPW

    def batch(g, _):
        off = base + g * K
        pltpu.sync_copy(src_hbm.at[pl.ds(off, K)], sidx)
        pltpu.sync_copy(dst_hbm.at[pl.ds(off, K)], didx)
        pltpu.sync_copy(alpha_hbm.at[pl.ds(off, K)], arows)
        pltpu.async_copy(h_hbm.at[sidx], rows, sem).wait()

        def scale(j, _):
            a = arows[j, :]
            for f in range(F // 16):
                rows[j, pl.ds(f * 16, 16)] = rows[j, pl.ds(f * 16, 16)] * a
            return 0
        lax.fori_loop(0, K, scale, 0)
        pltpu.sync_copy(rows, accum_sh.at[didx], add=True)
        return 0

    lax.fori_loop(0, NB, batch, 0)
    plsc.subcore_barrier()

    for cpy in range(RPT // RCH):
        r0 = sid * RPT + cpy * RCH
        pltpu.sync_copy(accum_sh.at[pl.ds(r0, RCH)], tmp)
        pltpu.sync_copy(tmp, out_hbm.at[cid, pl.ds(r0, RCH)])


@functools.partial(
    pl.kernel,
    mesh=plsc.VectorSubcoreMesh(core_axis_name="c", subcore_axis_name="s"),
    out_type=jax.ShapeDtypeStruct((NC, NPAD, F), jnp.float32),
    scratch_types=[
        pltpu.VMEM_SHARED((NPAD, F), jnp.float32),
        pltpu.VMEM((K,), jnp.int32),
        pltpu.VMEM((K,), jnp.int32),
        pltpu.VMEM((K, 16), jnp.float32),
        pltpu.VMEM((K, F), jnp.float32),
        pltpu.VMEM((RCH, F), jnp.float32),
        pltpu.SemaphoreType.DMA,
    ],
)
def _rowsum_sc(src_hbm, dst_hbm, alpha_hbm, h_hbm, out_hbm,
               accum_sh, sidx, didx, arows, rows, tmp, sem):
    _rowsum_body(src_hbm, dst_hbm, alpha_hbm, h_hbm, out_hbm,
                 accum_sh, sidx, didx, arows, rows, tmp, sem)


def _agg_rows(srcs, dsts, alphas16, h):
    dout = h.shape[1]
    parts = []
    for c0 in range(0, dout, F):
        parts.append(_rowsum_sc(srcs, dsts, alphas16, h[:, c0:c0 + F])[:, :N, :])
    out = jnp.concatenate(parts, axis=-1)
    return out[0], out[1]


def _edge_softmax(hs, hd, src, dst):
    e = jax.nn.leaky_relu(hs[src] + hd[dst], 0.2)
    m = jax.ops.segment_max(e, dst, num_segments=N)
    m = jnp.where(jnp.isfinite(m), m, 0.0)
    ex = jnp.exp(e - m[dst])
    s = jax.ops.segment_sum(ex, dst, num_segments=N)
    return ex / (s[dst] + 1e-16)


def kernel(x, edge_index, batch, W1, a_s1, a_d1, b1, W2, a_s2, a_d2, b2,
           W3, a_s3, a_d3, b3, fc1_W, fc1_b, g4, be4, fc2_W, fc2_b, g5, be5):
    src = edge_index[0]
    dst = edge_index[1]
    perm = jnp.argsort(dst, stable=True).astype(jnp.int32)
    srcs = src[perm]
    dsts = dst[perm]

    h, hs, hd = _lin_attn(x, W1, a_s1, a_d1)
    alpha = _edge_softmax(hs, hd, src, dst)
    a16 = jnp.broadcast_to(alpha[perm][:, None], (E, 16))
    p0, p1 = _agg_rows(srcs, dsts, a16, h)

    h, hs, hd = _epi_lin_attn(p0, p1, b1, W2, a_s2, a_d2)
    alpha = _edge_softmax(hs, hd, src, dst)
    a16 = jnp.broadcast_to(alpha[perm][:, None], (E, 16))
    p0, p1 = _agg_rows(srcs, dsts, a16, h)

    h, hs, hd = _epi_lin_attn(p0, p1, b2, W3, a_s3, a_d3)
    alpha = _edge_softmax(hs, hd, src, dst)
    a16 = jnp.broadcast_to(alpha[perm][:, None], (E, 16))
    p0, p1 = _agg_rows(srcs, dsts, a16, h)

    h3 = jax.nn.relu(p0 + p1 + b3)
    p = jax.ops.segment_max(h3, batch, num_segments=G)
    p = jnp.where(jnp.isfinite(p), p, 0.0)
    return _mlp(p, fc1_W, fc1_b, g4, be4, fc2_W, fc2_b, g5, be5)
